# baseline (speedup 1.0000x reference)
import functools
import jax
import jax.numpy as jnp
from jax import lax
from jax.experimental import pallas as pl
from jax.experimental.pallas import tpu as pltpu, tpu_sc as plsc

_NC, _NS = 2, 16
_NW = _NC * _NS
_R = 16
_L = 16


def kernel(x, pos_embedding):
    B, S, D = x.shape
    SW = S // _NW
    NT = SW // _R
    CH = D // _L
    mesh = plsc.VectorSubcoreMesh(core_axis_name="c", subcore_axis_name="s")

    @functools.partial(
        pl.kernel,
        mesh=mesh,
        out_type=jax.ShapeDtypeStruct((B * S, D), jnp.float32),
        scratch_types=[
            pltpu.VMEM((_R,), jnp.int32),
            pltpu.VMEM((_R, D), jnp.float32),
            pltpu.VMEM((_R, D), jnp.float32),
            pltpu.SemaphoreType.DMA,
        ],
    )
    def run(x_hbm, pos_hbm, out_hbm, idx_v, xt_v, pt_v, sem):
        wid = lax.axis_index("s") * _NC + lax.axis_index("c")
        s0 = wid * SW
        for t in range(NT):
            base = s0 + t * _R
            idx_v[...] = lax.iota(jnp.int32, _R) + base
            pltpu.async_copy(pos_hbm.at[idx_v], pt_v, sem).wait()
            for b in range(B):
                pltpu.sync_copy(x_hbm.at[pl.ds(b * S + base, _R)], xt_v)
                for r in range(_R):
                    def body(c, _):
                        sl = pl.ds(c * _L, _L)
                        plsc.addupdate(xt_v.at[r, sl], pt_v[r, sl])
                        return 0
                    lax.fori_loop(0, CH, body, 0)
                pltpu.sync_copy(xt_v, out_hbm.at[pl.ds(b * S + base, _R)])

    out = run(x.reshape(B * S, D), pos_embedding)
    return out.reshape(B, S, D)


# speedup vs baseline: 2.5044x; 2.5044x over previous
import functools
import jax
import jax.numpy as jnp
from jax import lax
from jax.experimental import pallas as pl
from jax.experimental.pallas import tpu as pltpu, tpu_sc as plsc

_NC, _NS = 2, 16
_NW = _NC * _NS
_R = 16
_L = 16


def kernel(x, pos_embedding):
    B, S, D = x.shape
    SW = S // _NW
    NT = SW // _R
    CH = D // _L
    mesh = plsc.VectorSubcoreMesh(core_axis_name="c", subcore_axis_name="s")

    @functools.partial(
        pl.kernel,
        mesh=mesh,
        out_type=jax.ShapeDtypeStruct((B * S, D), jnp.float32),
        scratch_types=[
            pltpu.VMEM((_R,), jnp.int32),
            pltpu.VMEM((2, _R, D), jnp.float32),
            pltpu.VMEM((_R, D), jnp.float32),
            pltpu.SemaphoreType.DMA,
            pltpu.SemaphoreType.DMA,
            pltpu.SemaphoreType.DMA,
            pltpu.SemaphoreType.DMA,
            pltpu.SemaphoreType.DMA,
        ],
    )
    def run(x_hbm, pos_hbm, out_hbm, idx_v, xt_v, pt_v, psem, li0, li1, so0, so1):
        lsem = (li0, li1)
        ssem = (so0, so1)
        wid = lax.axis_index("s") * _NC + lax.axis_index("c")
        s0 = wid * SW
        for t in range(NT):
            base = s0 + t * _R
            idx_v[...] = lax.iota(jnp.int32, _R) + base
            pgather = pltpu.async_copy(pos_hbm.at[idx_v], pt_v, psem)
            pltpu.async_copy(
                x_hbm.at[pl.ds(0 * S + base, _R)], xt_v.at[0], lsem[0]
            )
            pgather.wait()
            for b in range(B):
                cur = b % 2
                if b + 1 < B:
                    if b >= 1:
                        pltpu.make_async_copy(
                            xt_v.at[cur ^ 1], out_hbm.at[pl.ds((b - 1) * S + base, _R)], ssem[cur ^ 1]
                        ).wait()
                    pltpu.async_copy(
                        x_hbm.at[pl.ds((b + 1) * S + base, _R)], xt_v.at[cur ^ 1], lsem[cur ^ 1]
                    )
                pltpu.make_async_copy(
                    x_hbm.at[pl.ds(b * S + base, _R)], xt_v.at[cur], lsem[cur]
                ).wait()

                @plsc.parallel_loop(0, CH)
                def _(c):
                    for r in range(_R):
                        sl = pl.ds(c * _L, _L)
                        plsc.addupdate(xt_v.at[cur, r, sl], pt_v[r, sl])

                pltpu.async_copy(
                    xt_v.at[cur], out_hbm.at[pl.ds(b * S + base, _R)], ssem[cur]
                )
            pltpu.make_async_copy(
                xt_v.at[(B - 1) % 2], out_hbm.at[pl.ds((B - 1) * S + base, _R)], ssem[(B - 1) % 2]
            ).wait()
            pltpu.make_async_copy(
                xt_v.at[(B - 2) % 2], out_hbm.at[pl.ds((B - 2) * S + base, _R)], ssem[(B - 2) % 2]
            ).wait()

    out = run(x.reshape(B * S, D), pos_embedding)
    return out.reshape(B, S, D)


# speedup vs baseline: 4.4396x; 1.7727x over previous
import jax
import jax.numpy as jnp
from jax.experimental import pallas as pl
from jax.experimental.pallas import tpu as pltpu


def _add_body(x_ref, p_ref, o_ref):
    o_ref[...] = x_ref[...] + p_ref[...]


def kernel(x, pos_embedding):
    B, S, D = x.shape
    BS = 1024
    grid = (S // BS, B)
    return pl.pallas_call(
        _add_body,
        grid=grid,
        in_specs=[
            pl.BlockSpec((1, BS, D), lambda i, b: (b, i, 0)),
            pl.BlockSpec((BS, D), lambda i, b: (i, 0)),
        ],
        out_specs=pl.BlockSpec((1, BS, D), lambda i, b: (b, i, 0)),
        out_shape=jax.ShapeDtypeStruct((B, S, D), x.dtype),
        compiler_params=pltpu.CompilerParams(vmem_limit_bytes=120 * 1024 * 1024),
    )(x, pos_embedding)
